# Initial kernel scaffold; baseline (speedup 1.0000x reference)
import jax
import jax.numpy as jnp
from jax.experimental import pallas as pl


def kernel(x, edge_index, batch, W1, b1, W2, b2, Wl, bl):
    raise NotImplementedError("write your pallas kernel here")


# speedup vs baseline: 21.2795x; 21.2795x over previous
import functools

import jax
import jax.numpy as jnp
from jax import lax
from jax.experimental import pallas as pl
from jax.experimental.pallas import tpu as pltpu
from jax.experimental.pallas import tpu_sc as plsc

N_NODES = 10000
N_EDGES = 320000
IN_DIM = 128
HID = 64
EMB = 128
NUM_GRAPHS = 64

NC = 2
NS = 16
NW = NC * NS
E_TILE = N_EDGES // NW
CHUNK = 128
K_CH = -(-E_TILE // CHUNK)
E_TILE_P = K_CH * CHUNK
NP = 10240
DUMP = NP - 1
ROWS_T = NP // NS
WIDE = 128
E_VEC = E_TILE // 16

_MESH = plsc.VectorSubcoreMesh(core_axis_name="c", subcore_axis_name="s")
def _edge_chunks(edge_index):
    src_flat = edge_index[0].astype(jnp.int32).reshape(NW, E_TILE)
    dst_flat = edge_index[1].astype(jnp.int32).reshape(NW, E_TILE)
    pad = E_TILE_P - E_TILE
    src = jnp.concatenate(
        [src_flat, jnp.zeros((NW, pad), jnp.int32)], axis=1)
    dst = jnp.concatenate(
        [dst_flat, jnp.full((NW, pad), DUMP, jnp.int32)], axis=1)
    return (src.reshape(NC, NS, K_CH, CHUNK), dst.reshape(NC, NS, K_CH, CHUNK))


@functools.partial(
    pl.kernel,
    mesh=_MESH,
    out_type=jax.ShapeDtypeStruct((NW, NP), jnp.float32),
    scratch_types=[
        pltpu.VMEM((E_VEC, 16), jnp.int32),
        pltpu.VMEM((NP,), jnp.float32),
    ],
    compiler_params=pltpu.CompilerParams(needs_layout_passes=False),
)
def _deg_kernel(dst_hbm, out_hbm, dst_v, deg_local):
    c = lax.axis_index("c")
    s = lax.axis_index("s")
    pltpu.sync_copy(dst_hbm.at[c, s], dst_v)

    def zbody(j, carry):
        deg_local[pl.ds(j * 16, 16)] = jnp.zeros((16,), jnp.float32)
        return carry

    lax.fori_loop(0, NP // 16, zbody, 0)

    ones16 = jnp.ones((16,), jnp.float32)

    def body(j, carry):
        idx = dst_v[j]
        plsc.addupdate_scatter(deg_local, [idx], ones16)
        return carry

    lax.fori_loop(0, E_VEC, body, 0)
    pltpu.sync_copy(deg_local, out_hbm.at[c * NS + s])


@functools.partial(
    pl.kernel,
    mesh=_MESH,
    out_type=jax.ShapeDtypeStruct((NC, NP, HID), jnp.float32),
    scratch_types=[
        pltpu.VMEM((K_CH, CHUNK), jnp.int32),
        pltpu.VMEM((K_CH, CHUNK), jnp.int32),
        pltpu.VMEM((CHUNK,), jnp.int32),
        pltpu.VMEM((CHUNK,), jnp.int32),
        pltpu.VMEM((CHUNK, HID), jnp.float32),
        pltpu.VMEM((128, HID), jnp.float32),
        pltpu.VMEM_SHARED((NP, HID), jnp.float32),
        pltpu.SemaphoreType.DMA,
    ],
    compiler_params=pltpu.CompilerParams(use_tc_tiling_on_sc=False),
)
def _agg_kernel(p_hbm, src_hbm, dst_hbm, out_hbm,
                src_v, dst_v, icur, dcur, buf0, stage, acc, sem0):
    c = lax.axis_index("c")
    s = lax.axis_index("s")
    pltpu.sync_copy(src_hbm.at[c, s], src_v)
    pltpu.sync_copy(dst_hbm.at[c, s], dst_v)

    def zrow(i, carry):
        for k in range(HID // 16):
            stage[i, pl.ds(k * 16, 16)] = jnp.zeros((16,), jnp.float32)
        return carry

    lax.fori_loop(0, 128, zrow, 0)

    def zcp(m, carry):
        pltpu.sync_copy(stage, acc.at[pl.ds(s * ROWS_T + m * 128, 128)])
        return carry

    lax.fori_loop(0, ROWS_T // 128, zcp, 0)
    plsc.subcore_barrier()

    def body(j, carry):
        for k in range(CHUNK // 16):
            icur[pl.ds(k * 16, 16)] = src_v[j, pl.ds(k * 16, 16)]
            dcur[pl.ds(k * 16, 16)] = dst_v[j, pl.ds(k * 16, 16)]
        pltpu.async_copy(p_hbm.at[icur], buf0, sem0).wait()
        pltpu.sync_copy(buf0, acc.at[dcur], add=True)
        return carry

    lax.fori_loop(0, K_CH, body, 0)
    plsc.subcore_barrier()

    def wb(m, carry):
        pltpu.sync_copy(acc.at[pl.ds(s * ROWS_T + m * 128, 128)], stage)
        pltpu.sync_copy(stage, out_hbm.at[c, pl.ds(s * ROWS_T + m * 128, 128)])
        return carry

    lax.fori_loop(0, ROWS_T // 128, wb, 0)


def _mm1_body(cnt_ref, x_ref, w1_ref, dis_ref, p1s_ref):
    cnt = jnp.sum(cnt_ref[...], axis=0)[:N_NODES].reshape(N_NODES, 1)
    dis = lax.rsqrt(cnt + 1.0)
    p1 = jnp.dot(x_ref[...], w1_ref[...], preferred_element_type=jnp.float32)
    dis_ref[...] = dis
    p1s_ref[...] = dis * p1


def _mm2_body(t_ref, p1s_ref, dis_ref, b1_ref, w2_ref, p2s_ref):
    dis = dis_ref[...]
    t = (t_ref[0] + t_ref[1])[:N_NODES]
    h1 = jnp.maximum(dis * (t + p1s_ref[...]) + b1_ref[...], 0.0)
    p2 = jnp.dot(h1, w2_ref[...], preferred_element_type=jnp.float32)
    p2s_ref[...] = dis * p2


def _final_body(u_ref, p2s_ref, dis_ref, b2_ref, batch_ref, wl_ref, bl_ref,
                out_ref):
    dis = dis_ref[...]
    u = (u_ref[0] + u_ref[1])[:N_NODES]
    h2 = jnp.maximum(dis * (u + p2s_ref[...]) + b2_ref[...], 0.0)
    gid = lax.broadcasted_iota(jnp.int32, (NUM_GRAPHS, N_NODES), 0)
    onehot = (gid == batch_ref[...]).astype(jnp.float32)
    sums = jnp.dot(onehot, h2, preferred_element_type=jnp.float32)
    cnt = jnp.sum(onehot, axis=1, keepdims=True)
    pool = sums / jnp.maximum(cnt, 1.0)
    out_ref[...] = (jnp.dot(pool, wl_ref[...], preferred_element_type=jnp.float32)
                    + bl_ref[...])


def kernel(x, edge_index, batch, W1, b1, W2, b2, Wl, bl):
    src, dst = _edge_chunks(edge_index)
    dst_f = edge_index[1].astype(jnp.int32).reshape(NC, NS, E_VEC, 16)
    batch32 = batch.astype(jnp.int32).reshape(1, N_NODES)

    cnt = _deg_kernel(dst_f)

    dis, p1s = pl.pallas_call(
        _mm1_body,
        out_shape=[jax.ShapeDtypeStruct((N_NODES, 1), jnp.float32),
                   jax.ShapeDtypeStruct((N_NODES, HID), jnp.float32)],
    )(cnt, x, W1)

    t = _agg_kernel(p1s, src, dst)

    p2s = pl.pallas_call(
        _mm2_body,
        out_shape=jax.ShapeDtypeStruct((N_NODES, HID), jnp.float32),
    )(t, p1s, dis, b1, W2)

    u = _agg_kernel(p2s, src, dst)

    out = pl.pallas_call(
        _final_body,
        out_shape=jax.ShapeDtypeStruct((NUM_GRAPHS, EMB), jnp.float32),
    )(u, p2s, dis, b2, batch32, Wl, bl)
    return out


# speedup vs baseline: 26.8047x; 1.2596x over previous
import functools

import jax
import jax.numpy as jnp
from jax import lax
from jax.experimental import pallas as pl
from jax.experimental.pallas import tpu as pltpu
from jax.experimental.pallas import tpu_sc as plsc

N_NODES = 10000
N_EDGES = 320000
IN_DIM = 128
HID = 64
EMB = 128
NUM_GRAPHS = 64

NC = 2
NS = 16
NW = NC * NS
E_TILE = N_EDGES // NW
CHUNK = 128
K_CH = -(-E_TILE // CHUNK)
E_TILE_P = K_CH * CHUNK
NP = 10240
DUMP = NP - 1
ROWS_T = NP // NS
WIDE = 128
E_VEC = E_TILE // 16

_MESH = plsc.VectorSubcoreMesh(core_axis_name="c", subcore_axis_name="s")
def _edge_chunks(edge_index):
    src_flat = edge_index[0].astype(jnp.int32).reshape(NW, E_TILE)
    dst_flat = edge_index[1].astype(jnp.int32).reshape(NW, E_TILE)
    pad = E_TILE_P - E_TILE
    src = jnp.concatenate(
        [src_flat, jnp.zeros((NW, pad), jnp.int32)], axis=1)
    dst = jnp.concatenate(
        [dst_flat, jnp.full((NW, pad), DUMP, jnp.int32)], axis=1)
    return (src.reshape(NC, NS, K_CH, CHUNK), dst.reshape(NC, NS, K_CH, CHUNK))


@functools.partial(
    pl.kernel,
    mesh=_MESH,
    out_type=jax.ShapeDtypeStruct((NW, NP), jnp.float32),
    scratch_types=[
        pltpu.VMEM((E_VEC, 16), jnp.int32),
        pltpu.VMEM((NP,), jnp.float32),
    ],
    compiler_params=pltpu.CompilerParams(needs_layout_passes=False),
)
def _deg_kernel(dst_hbm, out_hbm, dst_v, deg_local):
    c = lax.axis_index("c")
    s = lax.axis_index("s")
    pltpu.sync_copy(dst_hbm.at[c, s], dst_v)

    def zbody(j, carry):
        deg_local[pl.ds(j * 16, 16)] = jnp.zeros((16,), jnp.float32)
        return carry

    lax.fori_loop(0, NP // 16, zbody, 0)

    ones16 = jnp.ones((16,), jnp.float32)

    def body(j, carry):
        idx = dst_v[j]
        plsc.addupdate_scatter(deg_local, [idx], ones16)
        return carry

    lax.fori_loop(0, E_VEC, body, 0)
    pltpu.sync_copy(deg_local, out_hbm.at[c * NS + s])


@functools.partial(
    pl.kernel,
    mesh=_MESH,
    out_type=jax.ShapeDtypeStruct((NC, NP, HID), jnp.float32),
    scratch_types=[
        pltpu.VMEM((K_CH, CHUNK), jnp.int32),
        pltpu.VMEM((K_CH, CHUNK), jnp.int32),
        pltpu.VMEM((CHUNK,), jnp.int32),
        pltpu.VMEM((CHUNK,), jnp.int32),
        pltpu.VMEM((CHUNK,), jnp.int32),
        pltpu.VMEM((CHUNK,), jnp.int32),
        pltpu.VMEM((CHUNK, HID), jnp.float32),
        pltpu.VMEM((CHUNK, HID), jnp.float32),
        pltpu.VMEM((128, HID), jnp.float32),
        pltpu.VMEM_SHARED((NP, HID), jnp.float32),
        pltpu.SemaphoreType.DMA,
        pltpu.SemaphoreType.DMA,
    ],
    compiler_params=pltpu.CompilerParams(use_tc_tiling_on_sc=False),
)
def _agg_kernel(p_hbm, src_hbm, dst_hbm, out_hbm,
                src_v, dst_v, icur0, dcur0, icur1, dcur1,
                buf0, buf1, stage, acc, sem0, sem1):
    c = lax.axis_index("c")
    s = lax.axis_index("s")
    pltpu.sync_copy(src_hbm.at[c, s], src_v)
    pltpu.sync_copy(dst_hbm.at[c, s], dst_v)

    def zrow(i, carry):
        for k in range(HID // 16):
            stage[i, pl.ds(k * 16, 16)] = jnp.zeros((16,), jnp.float32)
        return carry

    lax.fori_loop(0, 128, zrow, 0)

    def zcp(m, carry):
        pltpu.sync_copy(stage, acc.at[pl.ds(s * ROWS_T + m * 128, 128)])
        return carry

    lax.fori_loop(0, ROWS_T // 128, zcp, 0)
    plsc.subcore_barrier()

    def copy_idx(j, ic, dc):
        for k in range(CHUNK // 16):
            ic[pl.ds(k * 16, 16)] = src_v[j, pl.ds(k * 16, 16)]
            dc[pl.ds(k * 16, 16)] = dst_v[j, pl.ds(k * 16, 16)]

    copy_idx(0, icur0, dcur0)
    pltpu.make_async_copy(p_hbm.at[icur0], buf0, sem0).start()

    def body(jj, carry):
        j0 = 2 * jj
        copy_idx(j0 + 1, icur1, dcur1)
        pltpu.make_async_copy(p_hbm.at[icur1], buf1, sem1).start()
        pltpu.make_async_copy(p_hbm.at[icur0], buf0, sem0).wait()
        pltpu.sync_copy(buf0, acc.at[dcur0], add=True)
        copy_idx(j0 + 2, icur0, dcur0)
        pltpu.make_async_copy(p_hbm.at[icur0], buf0, sem0).start()
        pltpu.make_async_copy(p_hbm.at[icur1], buf1, sem1).wait()
        pltpu.sync_copy(buf1, acc.at[dcur1], add=True)
        return carry

    lax.fori_loop(0, K_CH // 2, body, 0)
    pltpu.make_async_copy(p_hbm.at[icur0], buf0, sem0).wait()
    pltpu.sync_copy(buf0, acc.at[dcur0], add=True)
    plsc.subcore_barrier()

    def wb(m, carry):
        pltpu.sync_copy(acc.at[pl.ds(s * ROWS_T + m * 128, 128)], stage)
        pltpu.sync_copy(stage, out_hbm.at[c, pl.ds(s * ROWS_T + m * 128, 128)])
        return carry

    lax.fori_loop(0, ROWS_T // 128, wb, 0)


def _mm1_body(cnt_ref, x_ref, w1_ref, dis_ref, p1s_ref):
    cnt = jnp.sum(cnt_ref[...], axis=0)[:N_NODES].reshape(N_NODES, 1)
    dis = lax.rsqrt(cnt + 1.0)
    p1 = jnp.dot(x_ref[...], w1_ref[...], preferred_element_type=jnp.float32)
    dis_ref[...] = dis
    p1s_ref[...] = dis * p1


def _mm2_body(t_ref, p1s_ref, dis_ref, b1_ref, w2_ref, p2s_ref):
    dis = dis_ref[...]
    t = (t_ref[0] + t_ref[1])[:N_NODES]
    h1 = jnp.maximum(dis * (t + p1s_ref[...]) + b1_ref[...], 0.0)
    p2 = jnp.dot(h1, w2_ref[...], preferred_element_type=jnp.float32)
    p2s_ref[...] = dis * p2


def _final_body(u_ref, p2s_ref, dis_ref, b2_ref, batch_ref, wl_ref, bl_ref,
                out_ref):
    dis = dis_ref[...]
    u = (u_ref[0] + u_ref[1])[:N_NODES]
    h2 = jnp.maximum(dis * (u + p2s_ref[...]) + b2_ref[...], 0.0)
    gid = lax.broadcasted_iota(jnp.int32, (NUM_GRAPHS, N_NODES), 0)
    onehot = (gid == batch_ref[...]).astype(jnp.float32)
    sums = jnp.dot(onehot, h2, preferred_element_type=jnp.float32)
    cnt = jnp.sum(onehot, axis=1, keepdims=True)
    pool = sums / jnp.maximum(cnt, 1.0)
    out_ref[...] = (jnp.dot(pool, wl_ref[...], preferred_element_type=jnp.float32)
                    + bl_ref[...])


def kernel(x, edge_index, batch, W1, b1, W2, b2, Wl, bl):
    src, dst = _edge_chunks(edge_index)
    dst_f = edge_index[1].astype(jnp.int32).reshape(NC, NS, E_VEC, 16)
    batch32 = batch.astype(jnp.int32).reshape(1, N_NODES)

    cnt = _deg_kernel(dst_f)

    dis, p1s = pl.pallas_call(
        _mm1_body,
        out_shape=[jax.ShapeDtypeStruct((N_NODES, 1), jnp.float32),
                   jax.ShapeDtypeStruct((N_NODES, HID), jnp.float32)],
    )(cnt, x, W1)

    t = _agg_kernel(p1s, src, dst)

    p2s = pl.pallas_call(
        _mm2_body,
        out_shape=jax.ShapeDtypeStruct((N_NODES, HID), jnp.float32),
    )(t, p1s, dis, b1, W2)

    u = _agg_kernel(p2s, src, dst)

    out = pl.pallas_call(
        _final_body,
        out_shape=jax.ShapeDtypeStruct((NUM_GRAPHS, EMB), jnp.float32),
    )(u, p2s, dis, b2, batch32, Wl, bl)
    return out


# speedup vs baseline: 27.7950x; 1.0369x over previous
import functools

import jax
import jax.numpy as jnp
from jax import lax
from jax.experimental import pallas as pl
from jax.experimental.pallas import tpu as pltpu
from jax.experimental.pallas import tpu_sc as plsc

N_NODES = 10000
N_EDGES = 320000
IN_DIM = 128
HID = 64
EMB = 128
NUM_GRAPHS = 64

NC = 2
NS = 16
NW = NC * NS
E_TILE = N_EDGES // NW
CHUNK = 128
K_CH = -(-E_TILE // CHUNK)
E_TILE_P = K_CH * CHUNK
NP = 10240
DUMP = NP - 1
ROWS_T = NP // NS
WIDE = 128
E_VEC = E_TILE // 16

_MESH = plsc.VectorSubcoreMesh(core_axis_name="c", subcore_axis_name="s")
def _edge_chunks(edge_index):
    src_flat = edge_index[0].astype(jnp.int32).reshape(NW, E_TILE)
    dst_flat = edge_index[1].astype(jnp.int32).reshape(NW, E_TILE)
    pad = E_TILE_P - E_TILE
    src = jnp.concatenate(
        [src_flat, jnp.zeros((NW, pad), jnp.int32)], axis=1)
    dst = jnp.concatenate(
        [dst_flat, jnp.full((NW, pad), DUMP, jnp.int32)], axis=1)
    return (src.reshape(NC, NS, K_CH, CHUNK), dst.reshape(NC, NS, K_CH, CHUNK))


@functools.partial(
    pl.kernel,
    mesh=_MESH,
    out_type=jax.ShapeDtypeStruct((NW, NP), jnp.float32),
    scratch_types=[
        pltpu.VMEM((E_VEC, 16), jnp.int32),
        pltpu.VMEM((NP,), jnp.float32),
    ],
    compiler_params=pltpu.CompilerParams(needs_layout_passes=False),
)
def _deg_kernel(dst_hbm, out_hbm, dst_v, deg_local):
    c = lax.axis_index("c")
    s = lax.axis_index("s")
    pltpu.sync_copy(dst_hbm.at[c, s], dst_v)

    def zbody(j, carry):
        deg_local[pl.ds(j * 16, 16)] = jnp.zeros((16,), jnp.float32)
        return carry

    lax.fori_loop(0, NP // 16, zbody, 0)

    ones16 = jnp.ones((16,), jnp.float32)

    def body(j, carry):
        idx = dst_v[j]
        plsc.addupdate_scatter(deg_local, [idx], ones16)
        return carry

    lax.fori_loop(0, E_VEC, body, 0)
    pltpu.sync_copy(deg_local, out_hbm.at[c * NS + s])


@functools.partial(
    pl.kernel,
    mesh=_MESH,
    out_type=jax.ShapeDtypeStruct((NC, NP, HID), jnp.float32),
    scratch_types=[
        pltpu.VMEM((K_CH, CHUNK), jnp.int32),
        pltpu.VMEM((K_CH, CHUNK), jnp.int32),
        pltpu.VMEM((4, CHUNK), jnp.int32),
        pltpu.VMEM((4, CHUNK), jnp.int32),
        pltpu.VMEM((CHUNK, HID), jnp.float32),
        pltpu.VMEM((CHUNK, HID), jnp.float32),
        pltpu.VMEM((CHUNK, HID), jnp.float32),
        pltpu.VMEM((CHUNK, HID), jnp.float32),
        pltpu.VMEM((128, HID), jnp.float32),
        pltpu.VMEM_SHARED((NP, HID), jnp.float32),
        pltpu.SemaphoreType.DMA,
        pltpu.SemaphoreType.DMA,
        pltpu.SemaphoreType.DMA,
        pltpu.SemaphoreType.DMA,
        pltpu.SemaphoreType.DMA,
        pltpu.SemaphoreType.DMA,
        pltpu.SemaphoreType.DMA,
        pltpu.SemaphoreType.DMA,
    ],
    compiler_params=pltpu.CompilerParams(use_tc_tiling_on_sc=False),
)
def _agg_kernel(p_hbm, src_hbm, dst_hbm, out_hbm,
                src_v, dst_v, ics, dcs, buf0, buf1, buf2, buf3, stage, acc,
                sg0, sg1, sg2, sg3, ss0, ss1, ss2, ss3):
    c = lax.axis_index("c")
    s = lax.axis_index("s")
    pltpu.sync_copy(src_hbm.at[c, s], src_v)
    pltpu.sync_copy(dst_hbm.at[c, s], dst_v)

    def zrow(i, carry):
        for k in range(HID // 16):
            stage[i, pl.ds(k * 16, 16)] = jnp.zeros((16,), jnp.float32)
        return carry

    lax.fori_loop(0, 128, zrow, 0)

    def zcp(m, carry):
        pltpu.sync_copy(stage, acc.at[pl.ds(s * ROWS_T + m * 128, 128)])
        return carry

    lax.fori_loop(0, ROWS_T // 128, zcp, 0)
    plsc.subcore_barrier()

    bufs = (buf0, buf1, buf2, buf3)
    sgs = (sg0, sg1, sg2, sg3)
    sss = (ss0, ss1, ss2, ss3)

    def copy_idx(j, b):
        for k in range(CHUNK // 16):
            ics[b, pl.ds(k * 16, 16)] = src_v[j, pl.ds(k * 16, 16)]
            dcs[b, pl.ds(k * 16, 16)] = dst_v[j, pl.ds(k * 16, 16)]

    def start_gather(b):
        pltpu.make_async_copy(p_hbm.at[ics.at[b]], bufs[b], sgs[b]).start()

    def wait_gather(b):
        pltpu.make_async_copy(p_hbm.at[ics.at[b]], bufs[b], sgs[b]).wait()

    def start_scatter(b):
        pltpu.make_async_copy(bufs[b], acc.at[dcs.at[b]], sss[b]).start()

    def wait_scatter(b):
        pltpu.make_async_copy(bufs[b], acc.at[dcs.at[b]], sss[b]).wait()

    def step(j, b, wait_prev_scatter):
        wait_gather(b)
        start_scatter(b)
        j2 = j + 2
        if j2 < K_CH:
            b2 = (b + 2) % 4
            if wait_prev_scatter:
                wait_scatter(b2)
            copy_idx(j2, b2)
            start_gather(b2)

    copy_idx(0, 0)
    start_gather(0)
    copy_idx(1, 1)
    start_gather(1)
    step(0, 0, False)
    step(1, 1, False)
    step(2, 2, True)
    step(3, 3, True)

    def body(jj, carry):
        j0 = 4 * jj
        for b in range(4):
            j = j0 + b
            wait_gather(b)
            start_scatter(b)
            b2 = (b + 2) % 4
            wait_scatter(b2)
            copy_idx(j + 2, b2)
            start_gather(b2)
        return carry

    lax.fori_loop(1, (K_CH - 3) // 4, body, 0)
    step(K_CH - 3, 0, True)
    step(K_CH - 2, 1, True)
    step(K_CH - 1, 2, True)
    wait_scatter(3)
    wait_scatter(0)
    wait_scatter(1)
    wait_scatter(2)
    plsc.subcore_barrier()

    def wb(m, carry):
        pltpu.sync_copy(acc.at[pl.ds(s * ROWS_T + m * 128, 128)], stage)
        pltpu.sync_copy(stage, out_hbm.at[c, pl.ds(s * ROWS_T + m * 128, 128)])
        return carry

    lax.fori_loop(0, ROWS_T // 128, wb, 0)


def _mm1_body(cnt_ref, x_ref, w1_ref, dis_ref, p1s_ref):
    cnt = jnp.sum(cnt_ref[...], axis=0)[:N_NODES].reshape(N_NODES, 1)
    dis = lax.rsqrt(cnt + 1.0)
    p1 = jnp.dot(x_ref[...], w1_ref[...], preferred_element_type=jnp.float32)
    dis_ref[...] = dis
    p1s_ref[...] = dis * p1


def _mm2_body(t_ref, p1s_ref, dis_ref, b1_ref, w2_ref, p2s_ref):
    dis = dis_ref[...]
    t = (t_ref[0] + t_ref[1])[:N_NODES]
    h1 = jnp.maximum(dis * (t + p1s_ref[...]) + b1_ref[...], 0.0)
    p2 = jnp.dot(h1, w2_ref[...], preferred_element_type=jnp.float32)
    p2s_ref[...] = dis * p2


def _final_body(u_ref, p2s_ref, dis_ref, b2_ref, batch_ref, wl_ref, bl_ref,
                out_ref):
    dis = dis_ref[...]
    u = (u_ref[0] + u_ref[1])[:N_NODES]
    h2 = jnp.maximum(dis * (u + p2s_ref[...]) + b2_ref[...], 0.0)
    gid = lax.broadcasted_iota(jnp.int32, (NUM_GRAPHS, N_NODES), 0)
    onehot = (gid == batch_ref[...]).astype(jnp.float32)
    sums = jnp.dot(onehot, h2, preferred_element_type=jnp.float32)
    cnt = jnp.sum(onehot, axis=1, keepdims=True)
    pool = sums / jnp.maximum(cnt, 1.0)
    out_ref[...] = (jnp.dot(pool, wl_ref[...], preferred_element_type=jnp.float32)
                    + bl_ref[...])


def kernel(x, edge_index, batch, W1, b1, W2, b2, Wl, bl):
    src, dst = _edge_chunks(edge_index)
    dst_f = edge_index[1].astype(jnp.int32).reshape(NC, NS, E_VEC, 16)
    batch32 = batch.astype(jnp.int32).reshape(1, N_NODES)

    cnt = _deg_kernel(dst_f)

    dis, p1s = pl.pallas_call(
        _mm1_body,
        out_shape=[jax.ShapeDtypeStruct((N_NODES, 1), jnp.float32),
                   jax.ShapeDtypeStruct((N_NODES, HID), jnp.float32)],
    )(cnt, x, W1)

    t = _agg_kernel(p1s, src, dst)

    p2s = pl.pallas_call(
        _mm2_body,
        out_shape=jax.ShapeDtypeStruct((N_NODES, HID), jnp.float32),
    )(t, p1s, dis, b1, W2)

    u = _agg_kernel(p2s, src, dst)

    out = pl.pallas_call(
        _final_body,
        out_shape=jax.ShapeDtypeStruct((NUM_GRAPHS, EMB), jnp.float32),
    )(u, p2s, dis, b2, batch32, Wl, bl)
    return out
